# baseline (speedup 1.0000x reference)
import jax
import jax.numpy as jnp
from jax.experimental import pallas as pl
from jax.experimental.pallas import tpu as pltpu
from jax.experimental.pallas import tpu_sc as plsc

_N, _D, _H, _E = 4096, 768, 3072, 8
_MU, _G1, _G2, _B1, _B2 = 0.7, 1.0, 1.0, 0.9, 0.999
_BLK = 256
_PMAX = _N * 2 + _E * _BLK
_NBLK = _PMAX // _BLK
_W = 128
_DH = _D // 2


_CB = 512
_NC = _N // _CB


def _route_meta_body(x_ref, wg_ref, bg_ref, posh_ref, p1_ref, p2_ref,
                     gb_ref, nblk_ref, i1_s, i2_s, rt_s):
    f32, i32 = jnp.float32, jnp.int32
    hi = jax.lax.Precision.HIGHEST

    rb = 1024
    for c in range(_N // rb):
        sl = slice(c * rb, (c + 1) * rb)
        logits = jax.lax.dot_general(
            x_ref[sl, :].astype(jnp.bfloat16),
            wg_ref[...].astype(jnp.bfloat16),
            (((1,), (0,)), ((), ())),
            preferred_element_type=f32) + bg_ref[...]
        lane = jax.lax.broadcasted_iota(i32, logits.shape, 1)
        i1 = jnp.argmax(logits, axis=1)[:, None]
        m1 = jnp.max(logits, axis=1, keepdims=True)
        masked = jnp.where(lane == i1, -1e30, logits)
        i2 = jnp.argmax(masked, axis=1)[:, None]
        m2 = jnp.max(masked, axis=1, keepdims=True)
        t = jnp.exp(m2 - m1)
        p1 = 1.0 / (1.0 + t)
        i1_s[sl, :] = i1
        i2_s[sl, :] = i2
        p1_ref[sl, :] = p1
        p2_ref[sl, :] = 1.0 - p1

    tril = (jax.lax.broadcasted_iota(i32, (_CB, _CB), 0)
            > jax.lax.broadcasted_iota(i32, (_CB, _CB), 1)).astype(jnp.bfloat16)
    eye = (jax.lax.broadcasted_iota(i32, (_CB, _CB), 0)
           == jax.lax.broadcasted_iota(i32, (_CB, _CB), 1)).astype(f32)

    def onehot(iref, c):
        idx = iref[c * _CB:(c + 1) * _CB, :]
        return (jax.lax.broadcasted_iota(i32, (_CB, _E), 1) == idx)

    base = jnp.zeros((1, _E), f32)
    for r, (iref, c) in enumerate([(i, c) for i in (i1_s, i2_s)
                                   for c in range(_NC)]):
        ohf = onehot(iref, c).astype(f32)
        prefix = jax.lax.dot_general(
            tril, ohf.astype(jnp.bfloat16), (((1,), (0,)), ((), ())),
            preferred_element_type=f32)
        rank = jnp.sum((prefix + base) * ohf, axis=1, keepdims=True)
        rt_s[r:r + 1, :] = jax.lax.dot_general(
            rank, eye, (((0,), (0,)), ((), ())), precision=hi)
        base = base + jnp.sum(ohf, axis=0, keepdims=True)

    counts = base.astype(i32)
    pc = ((counts + (_BLK - 1)) // _BLK) * _BLK
    pcf = pc.astype(f32)
    triu8 = (jax.lax.broadcasted_iota(i32, (_E, _E), 0)
             < jax.lax.broadcasted_iota(i32, (_E, _E), 1)).astype(f32)
    off = jax.lax.dot_general(
        pcf, triu8, (((1,), (0,)), ((), ())), precision=hi)

    dup = ((jax.lax.broadcasted_iota(i32, (_CB, 2 * _CB), 1) // 2)
           == jax.lax.broadcasted_iota(i32, (_CB, 2 * _CB), 0)).astype(f32)
    parity = (jax.lax.broadcasted_iota(i32, (1, 2 * _CB), 1) % 2).astype(f32)
    for r, (iref, c) in enumerate([(i, c) for i in (i1_s, i2_s)
                                   for c in range(_NC)]):
        ohf = onehot(iref, c).astype(f32)
        look = jnp.sum(ohf * off, axis=1, keepdims=True)
        lookt = jax.lax.dot_general(
            look, eye, (((0,), (0,)), ((), ())), precision=hi)
        post = rt_s[r:r + 1, :] + lookt
        row = 2.0 * jax.lax.dot_general(
            post, dup, (((1,), (0,)), ((), ())), precision=hi) + parity
        posh_ref[r:r + 1, :] = row.astype(i32)

    nblk_ref[...] = jnp.sum(pc // _BLK, axis=1, keepdims=True)
    blk_end = off + pcf
    eye8 = (jax.lax.broadcasted_iota(i32, (_E, _E), 0)
            == jax.lax.broadcasted_iota(i32, (_E, _E), 1)).astype(f32)
    blk_end_col = jax.lax.dot_general(
        eye8, blk_end, (((1,), (1,)), ((), ())), precision=hi)
    bi = (jax.lax.broadcasted_iota(i32, (_E, 128), 1) * _BLK).astype(f32)
    gbf = jnp.sum((bi >= blk_end_col).astype(f32), axis=0, keepdims=True)
    gb_ref[...] = jnp.minimum(gbf, _E - 1.0).astype(i32)


def _group_mm_body(gb_ref, nblk_ref, xs_ref, w1_ref, b1_ref, w2_ref, b2_ref,
                   ys_ref):
    del gb_ref
    b = pl.program_id(0)

    @pl.when(b < nblk_ref[0])
    def _():
        h = jnp.dot(xs_ref[...].astype(jnp.bfloat16), w1_ref[0],
                    preferred_element_type=jnp.float32)
        h = jnp.maximum(h + b1_ref[0], 0.0).astype(jnp.bfloat16)
        ys_ref[...] = jnp.dot(h, w2_ref[0],
                              preferred_element_type=jnp.float32) + b2_ref[0]


def _final_body(y0_ref, y1_ref, p1_ref, p2_ref, mom_ref, lnw_ref, lnb_ref,
                out_ref, p_ref, v_ref, mm_ref):
    mix = p1_ref[...] * y0_ref[...] + p2_ref[...] * y1_ref[...]
    mm = _MU * mom_ref[2] + _G2 * mix
    p = _B1 * mom_ref[0] + (1.0 - _B1) * mix
    v = _B2 * mom_ref[1] + (1.0 - _B2) * (mix * mix)
    y = -(_G1 / jnp.sqrt(v + 1e-8) * p)
    mean = jnp.mean(y, axis=1, keepdims=True)
    yc = y - mean
    var = jnp.mean(yc * yc, axis=1, keepdims=True)
    out_ref[...] = yc / jnp.sqrt(var + 1e-5) * lnw_ref[...] + lnb_ref[...]
    p_ref[...] = p
    v_ref[...] = v
    mm_ref[...] = mm


def _vmesh():
    return plsc.VectorSubcoreMesh(core_axis_name="c", subcore_axis_name="s")


def _dispatch(xh, posh):
    nsrc = 2 * _N // _W

    @pl.kernel(out_type=jax.ShapeDtypeStruct((2 * _PMAX, _DH), xh.dtype),
               mesh=_vmesh())
    def k(x_hbm, i_hbm, o_hbm):
        def body(x_vmem, i_vmem):
            pltpu.sync_copy(x_vmem, o_hbm.at[i_vmem.at[0]])

        pltpu.emit_pipeline(
            body,
            grid=(4 * _N // _W,),
            in_specs=[
                pl.BlockSpec((_W, _DH), lambda i: (jax.lax.rem(i, nsrc), 0)),
                pl.BlockSpec((1, _W), lambda i: (i // 8, i % 8)),
            ],
            out_specs=[],
            core_axis_name=("c", "s"),
            dimension_semantics=(pltpu.PARALLEL,),
        )(x_hbm, i_hbm)

    return k(xh, posh)


def _combine(ys2, posh):
    @pl.kernel(out_type=jax.ShapeDtypeStruct((4 * _N, _DH), ys2.dtype),
               mesh=_vmesh())
    def k(ys_hbm, i_hbm, o_hbm):
        def body(i_vmem, o_vmem):
            pltpu.sync_copy(ys_hbm.at[i_vmem.at[0]], o_vmem)

        pltpu.emit_pipeline(
            body,
            grid=(4 * _N // _W,),
            in_specs=[pl.BlockSpec((1, _W), lambda i: (i // 8, i % 8))],
            out_specs=[pl.BlockSpec((_W, _DH), lambda i: (i, 0))],
            core_axis_name=("c", "s"),
            dimension_semantics=(pltpu.PARALLEL,),
        )(i_hbm, o_hbm)

    return k(ys2, posh)


@jax.jit
def kernel(x, moment, W1, b1, W2, b2, Wg, bg, ln_w, ln_b):
    f32, i32 = jnp.float32, jnp.int32
    posh, p1, p2, gb, nblk = pl.pallas_call(
        _route_meta_body,
        grid=(1,),
        in_specs=[
            pl.BlockSpec((_N, _D), lambda i: (0, 0)),
            pl.BlockSpec((_D, _E), lambda i: (0, 0)),
            pl.BlockSpec((1, _E), lambda i: (0, 0)),
        ],
        out_specs=[pl.BlockSpec((2 * _NC, 2 * _CB), lambda i: (0, 0)),
                   pl.BlockSpec((_N, 1), lambda i: (0, 0)),
                   pl.BlockSpec((_N, 1), lambda i: (0, 0)),
                   pl.BlockSpec((1, 128), lambda i: (0, 0)),
                   pl.BlockSpec((1, 1), lambda i: (0, 0))],
        out_shape=[jax.ShapeDtypeStruct((2 * _NC, 2 * _CB), i32),
                   jax.ShapeDtypeStruct((_N, 1), f32),
                   jax.ShapeDtypeStruct((_N, 1), f32),
                   jax.ShapeDtypeStruct((1, 128), i32),
                   jax.ShapeDtypeStruct((1, 1), i32)],
        scratch_shapes=[pltpu.VMEM((_N, 1), i32),
                        pltpu.VMEM((_N, 1), i32),
                        pltpu.VMEM((2 * _NC, _CB), f32)],
        compiler_params=pltpu.CompilerParams(
            dimension_semantics=("arbitrary",)),
    )(x, Wg, bg.reshape(1, _E))

    xs = _dispatch(x.reshape(2 * _N, _DH), posh).reshape(_PMAX, _D)

    w1b = W1.astype(jnp.bfloat16)
    w2b = W2.astype(jnp.bfloat16)

    def _clamped(b, nb):
        return jnp.minimum(b, nb[0] - 1)

    ys = pl.pallas_call(
        _group_mm_body,
        grid_spec=pltpu.PrefetchScalarGridSpec(
            num_scalar_prefetch=2,
            grid=(_NBLK,),
            in_specs=[
                pl.BlockSpec((_BLK, _D),
                             lambda b, gb, nb: (_clamped(b, nb), 0)),
                pl.BlockSpec((1, _D, _H),
                             lambda b, gb, nb: (gb[_clamped(b, nb)], 0, 0)),
                pl.BlockSpec((1, 1, _H),
                             lambda b, gb, nb: (gb[_clamped(b, nb)], 0, 0)),
                pl.BlockSpec((1, _H, _D),
                             lambda b, gb, nb: (gb[_clamped(b, nb)], 0, 0)),
                pl.BlockSpec((1, 1, _D),
                             lambda b, gb, nb: (gb[_clamped(b, nb)], 0, 0)),
            ],
            out_specs=pl.BlockSpec((_BLK, _D),
                                   lambda b, gb, nb: (_clamped(b, nb), 0)),
        ),
        out_shape=jax.ShapeDtypeStruct((_PMAX, _D), f32),
        compiler_params=pltpu.CompilerParams(
            dimension_semantics=("arbitrary",)),
    )(gb.reshape(128), nblk.reshape(1), xs, w1b,
      b1.reshape(_E, 1, _H), w2b, b2.reshape(_E, 1, _D))

    y01 = _combine(ys.reshape(2 * _PMAX, _DH), posh).reshape(2 * _N, _D)

    bn = 512
    shp = jax.ShapeDtypeStruct((_N, _D), f32)
    out, p, v, mm = pl.pallas_call(
        _final_body,
        grid=(_N // bn,),
        in_specs=[
            pl.BlockSpec((bn, _D), lambda n: (n, 0)),
            pl.BlockSpec((bn, _D), lambda n: (n + _N // bn, 0)),
            pl.BlockSpec((bn, 1), lambda n: (n, 0)),
            pl.BlockSpec((bn, 1), lambda n: (n, 0)),
            pl.BlockSpec((3, bn, _D), lambda n: (0, n, 0)),
            pl.BlockSpec((1, _D), lambda n: (0, 0)),
            pl.BlockSpec((1, _D), lambda n: (0, 0)),
        ],
        out_specs=[pl.BlockSpec((bn, _D), lambda n: (n, 0))] * 4,
        out_shape=[shp, shp, shp, shp],
    )(y01, y01, p1, p2, moment, ln_w.reshape(1, _D), ln_b.reshape(1, _D))
    return (out, p, v, mm)


# speedup vs baseline: 1.1231x; 1.1231x over previous
import jax
import jax.numpy as jnp
from jax.experimental import pallas as pl
from jax.experimental.pallas import tpu as pltpu
from jax.experimental.pallas import tpu_sc as plsc

_N, _D, _H, _E = 4096, 768, 3072, 8
_MU, _G1, _G2, _B1, _B2 = 0.7, 1.0, 1.0, 0.9, 0.999
_BLK = 256
_PMAX = _N * 2 + _E * _BLK
_NBLK = _PMAX // _BLK
_W = 128
_DH = _D // 2


_CB = 512
_NC = _N // _CB
_STR = _PMAX // (2 * _CB)


def _route_meta_body(x_ref, wg_ref, bg_ref, posq_ref,
                     p1_ref, p2_ref, gb_ref, nblk_ref,
                     i1_s, i2_s, rank_s):
    f32, i32 = jnp.float32, jnp.int32
    hi = jax.lax.Precision.HIGHEST

    rb = 1024
    for c in range(_N // rb):
        sl = slice(c * rb, (c + 1) * rb)
        logits = jax.lax.dot_general(
            x_ref[sl, :].astype(jnp.bfloat16),
            wg_ref[...].astype(jnp.bfloat16),
            (((1,), (0,)), ((), ())),
            preferred_element_type=f32) + bg_ref[...]
        lane = jax.lax.broadcasted_iota(i32, logits.shape, 1)
        i1 = jnp.argmax(logits, axis=1)[:, None]
        m1 = jnp.max(logits, axis=1, keepdims=True)
        masked = jnp.where(lane == i1, -1e30, logits)
        i2 = jnp.argmax(masked, axis=1)[:, None]
        m2 = jnp.max(masked, axis=1, keepdims=True)
        t = jnp.exp(m2 - m1)
        p1 = 1.0 / (1.0 + t)
        i1_s[sl, :] = i1
        i2_s[sl, :] = i2
        p1_ref[sl, :] = p1
        p2_ref[sl, :] = 1.0 - p1

    tril = (jax.lax.broadcasted_iota(i32, (_CB, _CB), 0)
            > jax.lax.broadcasted_iota(i32, (_CB, _CB), 1)).astype(jnp.bfloat16)
    eye = (jax.lax.broadcasted_iota(i32, (_CB, _CB), 0)
           == jax.lax.broadcasted_iota(i32, (_CB, _CB), 1)).astype(jnp.bfloat16)

    def onehot(iref, c):
        idx = iref[c * _CB:(c + 1) * _CB, :]
        return (jax.lax.broadcasted_iota(i32, (_CB, _E), 1) == idx)

    base = jnp.zeros((1, _E), f32)
    for k, iref in enumerate((i1_s, i2_s)):
        for c in range(_NC):
            ohf = onehot(iref, c).astype(f32)
            prefix = jax.lax.dot_general(
                tril, ohf.astype(jnp.bfloat16), (((1,), (0,)), ((), ())),
                preferred_element_type=f32)
            rank = jnp.sum((prefix + base) * ohf, axis=1, keepdims=True)
            rank_s[c * _CB:(c + 1) * _CB, k:k + 1] = rank
            base = base + jnp.sum(ohf, axis=0, keepdims=True)

    counts = base.astype(i32)
    pc = ((counts + (_BLK - 1)) // _BLK) * _BLK
    pcf = pc.astype(f32)
    triu8 = (jax.lax.broadcasted_iota(i32, (_E, _E), 0)
             < jax.lax.broadcasted_iota(i32, (_E, _E), 1)).astype(f32)
    off = jax.lax.dot_general(
        pcf, triu8, (((1,), (0,)), ((), ())), precision=hi)

    def dotT(col, mat):
        ci = col.astype(i32)
        chi = (ci // 128).astype(jnp.bfloat16)
        clo = (ci % 128).astype(jnp.bfloat16)
        d = lambda a: jax.lax.dot_general(
            a, mat, (((0,), (0,)), ((), ())),
            preferred_element_type=f32)
        return 128.0 * d(chi) + d(clo)

    for k, iref in enumerate((i1_s, i2_s)):
        for c in range(_NC):
            ohf = onehot(iref, c).astype(f32)
            look = jnp.sum(ohf * off, axis=1, keepdims=True)
            pos_col = rank_s[c * _CB:(c + 1) * _CB, k:k + 1] + look
            post = dotT(pos_col, eye)
            r2 = k * _NC + c
            posq_ref[r2 // 2:r2 // 2 + 1,
                     (r2 % 2) * _CB:(r2 % 2 + 1) * _CB] = post.astype(i32)

    nblk_ref[...] = jnp.sum(pc // _BLK, axis=1, keepdims=True)
    blk_end = off + pcf
    eye8 = (jax.lax.broadcasted_iota(i32, (_E, _E), 0)
            == jax.lax.broadcasted_iota(i32, (_E, _E), 1)).astype(f32)
    blk_end_col = jax.lax.dot_general(
        eye8, blk_end, (((1,), (1,)), ((), ())), precision=hi)
    bi = (jax.lax.broadcasted_iota(i32, (_E, 128), 1) * _BLK).astype(f32)
    gbf = jnp.sum((bi >= blk_end_col).astype(f32), axis=0, keepdims=True)
    gb_ref[...] = jnp.minimum(gbf, _E - 1.0).astype(i32)


def _group_mm_body(gb_ref, nblk_ref, xlo_ref, xhi_ref, w1_ref, b1_ref,
                   w2_ref, b2_ref, ylo_ref, yhi_ref):
    del gb_ref
    b = pl.program_id(0)
    hh = pl.program_id(1)

    @pl.when(b < nblk_ref[0])
    def _():
        xs = jnp.concatenate([xlo_ref[...], xhi_ref[...]],
                             axis=1).astype(jnp.bfloat16)
        hid = jnp.dot(xs, w1_ref[0].astype(jnp.bfloat16),
                      preferred_element_type=jnp.float32)
        hid = jnp.maximum(hid + b1_ref[0], 0.0).astype(jnp.bfloat16)
        part = jnp.dot(hid, w2_ref[0].astype(jnp.bfloat16),
                       preferred_element_type=jnp.float32)

        @pl.when(hh == 0)
        def _():
            ylo_ref[...] = part[:, :_DH]
            yhi_ref[...] = part[:, _DH:]

        @pl.when(hh == 1)
        def _():
            ylo_ref[...] += part[:, :_DH] + b2_ref[0][:, :_DH]
            yhi_ref[...] += part[:, _DH:] + b2_ref[0][:, _DH:]


def _final_body(y0lo_ref, y0hi_ref, y1lo_ref, y1hi_ref, p1_ref, p2_ref,
                mom_ref, lnw_ref, lnb_ref, out_ref, p_ref, v_ref, mm_ref):
    y0 = jnp.concatenate([y0lo_ref[...], y0hi_ref[...]], axis=1)
    y1 = jnp.concatenate([y1lo_ref[...], y1hi_ref[...]], axis=1)
    mix = p1_ref[...] * y0 + p2_ref[...] * y1
    mm = _MU * mom_ref[2] + _G2 * mix
    p = _B1 * mom_ref[0] + (1.0 - _B1) * mix
    v = _B2 * mom_ref[1] + (1.0 - _B2) * (mix * mix)
    y = -(_G1 / jnp.sqrt(v + 1e-8) * p)
    mean = jnp.mean(y, axis=1, keepdims=True)
    yc = y - mean
    var = jnp.mean(yc * yc, axis=1, keepdims=True)
    out_ref[...] = yc / jnp.sqrt(var + 1e-5) * lnw_ref[...] + lnb_ref[...]
    p_ref[...] = p
    v_ref[...] = v
    mm_ref[...] = mm


def _vmesh():
    return plsc.VectorSubcoreMesh(core_axis_name="c", subcore_axis_name="s")


def _sc_scatter(data, idx):
    nsrc = _N // _W
    nlane = idx.shape[1] // _W

    @pl.kernel(out_type=jax.ShapeDtypeStruct((_PMAX, _DH), data.dtype),
               mesh=_vmesh())
    def k(d_hbm, i_hbm, o_hbm):
        def body(d_vmem, i_vmem):
            pltpu.sync_copy(d_vmem, o_hbm.at[i_vmem.at[0]])

        pltpu.emit_pipeline(
            body,
            grid=(2 * _N // _W,),
            in_specs=[
                pl.BlockSpec((_W, _DH), lambda i: (jax.lax.rem(i, nsrc), 0)),
                pl.BlockSpec(
                    (1, _W), lambda i: (i // nlane, jax.lax.rem(i, nlane))),
            ],
            out_specs=[],
            core_axis_name=("c", "s"),
            dimension_semantics=(pltpu.PARALLEL,),
        )(d_hbm, i_hbm)

    return k(data, idx)


def _sc_gather(data, idx, nrows):
    nlane = idx.shape[1] // _W

    @pl.kernel(out_type=jax.ShapeDtypeStruct((nrows, _DH), data.dtype),
               mesh=_vmesh())
    def k(d_hbm, i_hbm, o_hbm):
        def body(i_vmem, o_vmem):
            pltpu.sync_copy(d_hbm.at[i_vmem.at[0]], o_vmem)

        pltpu.emit_pipeline(
            body,
            grid=(nrows // _W,),
            in_specs=[pl.BlockSpec(
                (1, _W), lambda i: (i // nlane, jax.lax.rem(i, nlane)))],
            out_specs=[pl.BlockSpec((_W, _DH), lambda i: (i, 0))],
            core_axis_name=("c", "s"),
            dimension_semantics=(pltpu.PARALLEL,),
        )(i_hbm, o_hbm)

    return k(data, idx)


@jax.jit
def kernel(x, moment, W1, b1, W2, b2, Wg, bg, ln_w, ln_b):
    f32, i32 = jnp.float32, jnp.int32
    posq, p1, p2, gb, nblk = pl.pallas_call(
        _route_meta_body,
        grid=(1,),
        in_specs=[
            pl.BlockSpec((_N, _D), lambda i: (0, 0)),
            pl.BlockSpec((_D, _E), lambda i: (0, 0)),
            pl.BlockSpec((1, _E), lambda i: (0, 0)),
        ],
        out_specs=[pl.BlockSpec((_NC, 2 * _CB), lambda i: (0, 0)),
                   pl.BlockSpec((_N, 1), lambda i: (0, 0)),
                   pl.BlockSpec((_N, 1), lambda i: (0, 0)),
                   pl.BlockSpec((1, 128), lambda i: (0, 0)),
                   pl.BlockSpec((1, 1), lambda i: (0, 0))],
        out_shape=[jax.ShapeDtypeStruct((_NC, 2 * _CB), i32),
                   jax.ShapeDtypeStruct((_N, 1), f32),
                   jax.ShapeDtypeStruct((_N, 1), f32),
                   jax.ShapeDtypeStruct((1, 128), i32),
                   jax.ShapeDtypeStruct((1, 1), i32)],
        scratch_shapes=[pltpu.VMEM((_N, 1), i32),
                        pltpu.VMEM((_N, 1), i32),
                        pltpu.VMEM((_N, 2), f32)],
        compiler_params=pltpu.CompilerParams(
            dimension_semantics=("arbitrary",)),
    )(x, Wg, bg.reshape(1, _E))

    xlo = jax.lax.slice(x, (0, 0), (_N, _DH))
    xhi = jax.lax.slice(x, (0, _DH), (_N, _D))
    xslo = _sc_scatter(xlo, posq)
    xshi = _sc_scatter(xhi, posq)

    def _clamped(b, nb):
        return jnp.minimum(b, nb[0] - 1)

    yshp = jax.ShapeDtypeStruct((_PMAX, _DH), f32)
    yslo, yshi = pl.pallas_call(
        _group_mm_body,
        grid_spec=pltpu.PrefetchScalarGridSpec(
            num_scalar_prefetch=2,
            grid=(_NBLK, 2),
            in_specs=[
                pl.BlockSpec((_BLK, _DH),
                             lambda b, h, gb, nb: (_clamped(b, nb), 0)),
                pl.BlockSpec((_BLK, _DH),
                             lambda b, h, gb, nb: (_clamped(b, nb), 0)),
                pl.BlockSpec((1, _D, _H // 2),
                             lambda b, h, gb, nb:
                             (gb[_clamped(b, nb)], 0, h)),
                pl.BlockSpec((1, 1, _H // 2),
                             lambda b, h, gb, nb:
                             (gb[_clamped(b, nb)], 0, h)),
                pl.BlockSpec((1, _H // 2, _D),
                             lambda b, h, gb, nb:
                             (gb[_clamped(b, nb)], h, 0)),
                pl.BlockSpec((1, 1, _D),
                             lambda b, h, gb, nb:
                             (gb[_clamped(b, nb)], 0, 0)),
            ],
            out_specs=[
                pl.BlockSpec((_BLK, _DH),
                             lambda b, h, gb, nb: (_clamped(b, nb), 0)),
                pl.BlockSpec((_BLK, _DH),
                             lambda b, h, gb, nb: (_clamped(b, nb), 0)),
            ],
        ),
        out_shape=[yshp, yshp],
        compiler_params=pltpu.CompilerParams(
            dimension_semantics=("arbitrary", "arbitrary")),
    )(gb.reshape(128), nblk.reshape(1), xslo, xshi, W1,
      b1.reshape(_E, 1, _H), W2, b2.reshape(_E, 1, _D))

    y01lo = _sc_gather(yslo, posq, 2 * _N)
    y01hi = _sc_gather(yshi, posq, 2 * _N)

    bn = 512
    shp = jax.ShapeDtypeStruct((_N, _D), f32)
    out, p, v, mm = pl.pallas_call(
        _final_body,
        grid=(_N // bn,),
        in_specs=[
            pl.BlockSpec((bn, _DH), lambda n: (n, 0)),
            pl.BlockSpec((bn, _DH), lambda n: (n, 0)),
            pl.BlockSpec((bn, _DH), lambda n: (n + _N // bn, 0)),
            pl.BlockSpec((bn, _DH), lambda n: (n + _N // bn, 0)),
            pl.BlockSpec((bn, 1), lambda n: (n, 0)),
            pl.BlockSpec((bn, 1), lambda n: (n, 0)),
            pl.BlockSpec((3, bn, _D), lambda n: (0, n, 0)),
            pl.BlockSpec((1, _D), lambda n: (0, 0)),
            pl.BlockSpec((1, _D), lambda n: (0, 0)),
        ],
        out_specs=[pl.BlockSpec((bn, _D), lambda n: (n, 0))] * 4,
        out_shape=[shp, shp, shp, shp],
    )(y01lo, y01hi, y01lo, y01hi, p1, p2, moment,
      ln_w.reshape(1, _D), ln_b.reshape(1, _D))
    return (out, p, v, mm)


# speedup vs baseline: 1.5096x; 1.3441x over previous
import jax
import jax.numpy as jnp
from jax.experimental import pallas as pl
from jax.experimental.pallas import tpu as pltpu
from jax.experimental.pallas import tpu_sc as plsc

_N, _D, _H, _E = 4096, 768, 3072, 8
_MU, _G1, _G2, _B1, _B2 = 0.7, 1.0, 1.0, 0.9, 0.999
_BLK = 256
_PMAX = _N * 2 + _E * _BLK
_NBLK = _PMAX // _BLK
_W = 128
_DH = _D // 2


_CB = 512
_NC = _N // _CB
_STR = _PMAX // (2 * _CB)


def _route_meta_body(x_ref, wg_ref, bg_ref, posq_ref,
                     p1_ref, p2_ref, gb_ref, nblk_ref,
                     i1_s, i2_s, rank_s):
    f32, i32 = jnp.float32, jnp.int32
    hi = jax.lax.Precision.HIGHEST

    rb = 1024
    for c in range(_N // rb):
        sl = slice(c * rb, (c + 1) * rb)
        logits = jax.lax.dot_general(
            x_ref[sl, :].astype(jnp.bfloat16),
            wg_ref[...].astype(jnp.bfloat16),
            (((1,), (0,)), ((), ())),
            preferred_element_type=f32) + bg_ref[...]
        lane = jax.lax.broadcasted_iota(i32, logits.shape, 1)
        i1 = jnp.argmax(logits, axis=1)[:, None]
        m1 = jnp.max(logits, axis=1, keepdims=True)
        masked = jnp.where(lane == i1, -1e30, logits)
        i2 = jnp.argmax(masked, axis=1)[:, None]
        m2 = jnp.max(masked, axis=1, keepdims=True)
        t = jnp.exp(m2 - m1)
        p1 = 1.0 / (1.0 + t)
        i1_s[sl, :] = i1
        i2_s[sl, :] = i2
        p1_ref[sl, :] = p1
        p2_ref[sl, :] = 1.0 - p1

    tril = (jax.lax.broadcasted_iota(i32, (_CB, _CB), 0)
            > jax.lax.broadcasted_iota(i32, (_CB, _CB), 1)).astype(jnp.bfloat16)
    eye = (jax.lax.broadcasted_iota(i32, (_CB, _CB), 0)
           == jax.lax.broadcasted_iota(i32, (_CB, _CB), 1)).astype(jnp.bfloat16)

    def onehot(iref, c):
        idx = iref[c * _CB:(c + 1) * _CB, :]
        return (jax.lax.broadcasted_iota(i32, (_CB, _E), 1) == idx)

    base = jnp.zeros((1, _E), f32)
    for k, iref in enumerate((i1_s, i2_s)):
        for c in range(_NC):
            ohf = onehot(iref, c).astype(f32)
            prefix = jax.lax.dot_general(
                tril, ohf.astype(jnp.bfloat16), (((1,), (0,)), ((), ())),
                preferred_element_type=f32)
            rank = jnp.sum((prefix + base) * ohf, axis=1, keepdims=True)
            rank_s[c * _CB:(c + 1) * _CB, k:k + 1] = rank
            base = base + jnp.sum(ohf, axis=0, keepdims=True)

    counts = base.astype(i32)
    pc = ((counts + (_BLK - 1)) // _BLK) * _BLK
    pcf = pc.astype(f32)
    triu8 = (jax.lax.broadcasted_iota(i32, (_E, _E), 0)
             < jax.lax.broadcasted_iota(i32, (_E, _E), 1)).astype(f32)
    off = jax.lax.dot_general(
        pcf, triu8, (((1,), (0,)), ((), ())), precision=hi)

    def dotT(col, mat):
        ci = col.astype(i32)
        chi = (ci // 128).astype(jnp.bfloat16)
        clo = (ci % 128).astype(jnp.bfloat16)
        d = lambda a: jax.lax.dot_general(
            a, mat, (((0,), (0,)), ((), ())),
            preferred_element_type=f32)
        return 128.0 * d(chi) + d(clo)

    for k, iref in enumerate((i1_s, i2_s)):
        for c in range(_NC):
            ohf = onehot(iref, c).astype(f32)
            look = jnp.sum(ohf * off, axis=1, keepdims=True)
            pos_col = rank_s[c * _CB:(c + 1) * _CB, k:k + 1] + look
            post = dotT(pos_col, eye)
            r2 = k * _NC + c
            posq_ref[r2 // 2:r2 // 2 + 1,
                     (r2 % 2) * _CB:(r2 % 2 + 1) * _CB] = post.astype(i32)

    nblk_ref[...] = jnp.sum(pc // _BLK, axis=1, keepdims=True)
    blk_end = off + pcf
    eye8 = (jax.lax.broadcasted_iota(i32, (_E, _E), 0)
            == jax.lax.broadcasted_iota(i32, (_E, _E), 1)).astype(f32)
    blk_end_col = jax.lax.dot_general(
        eye8, blk_end, (((1,), (1,)), ((), ())), precision=hi)
    bi = (jax.lax.broadcasted_iota(i32, (_E, 128), 1) * _BLK).astype(f32)
    gbf = jnp.sum((bi >= blk_end_col).astype(f32), axis=0, keepdims=True)
    gb_ref[...] = jnp.minimum(gbf, _E - 1.0).astype(i32)


def _group_mm_body(gb_ref, nblk_ref, xlo_ref, xhi_ref, w1_ref, b1_ref,
                   w2_ref, b2_ref, ylo_ref, yhi_ref):
    del gb_ref
    b = pl.program_id(0)

    @pl.when(b < nblk_ref[0])
    def _():
        xs = jnp.concatenate([xlo_ref[...], xhi_ref[...]],
                             axis=1).astype(jnp.bfloat16)
        hid = jnp.dot(xs, w1_ref[0].astype(jnp.bfloat16),
                      preferred_element_type=jnp.float32)
        hid = jnp.maximum(hid + b1_ref[0], 0.0).astype(jnp.bfloat16)
        eo = jnp.dot(hid, w2_ref[0].astype(jnp.bfloat16),
                     preferred_element_type=jnp.float32) + b2_ref[0]
        ylo_ref[...] = eo[:, :_DH]
        yhi_ref[...] = eo[:, _DH:]


def _final_body(y0lo_ref, y0hi_ref, y1lo_ref, y1hi_ref, p1_ref, p2_ref,
                mom_ref, lnw_ref, lnb_ref, out_ref, p_ref, v_ref, mm_ref):
    y0 = jnp.concatenate([y0lo_ref[...], y0hi_ref[...]], axis=1)
    y1 = jnp.concatenate([y1lo_ref[...], y1hi_ref[...]], axis=1)
    mix = p1_ref[...] * y0 + p2_ref[...] * y1
    mm = _MU * mom_ref[2] + _G2 * mix
    p = _B1 * mom_ref[0] + (1.0 - _B1) * mix
    v = _B2 * mom_ref[1] + (1.0 - _B2) * (mix * mix)
    y = -(_G1 / jnp.sqrt(v + 1e-8) * p)
    mean = jnp.mean(y, axis=1, keepdims=True)
    yc = y - mean
    var = jnp.mean(yc * yc, axis=1, keepdims=True)
    out_ref[...] = yc / jnp.sqrt(var + 1e-5) * lnw_ref[...] + lnb_ref[...]
    p_ref[...] = p
    v_ref[...] = v
    mm_ref[...] = mm


def _vmesh():
    return plsc.VectorSubcoreMesh(core_axis_name="c", subcore_axis_name="s")


def _sc_scatter(data, idx):
    nsrc = _N // _W
    nlane = idx.shape[1] // _W

    @pl.kernel(out_type=jax.ShapeDtypeStruct((_PMAX, _DH), data.dtype),
               mesh=_vmesh())
    def k(d_hbm, i_hbm, o_hbm):
        def body(d_vmem, i_vmem):
            pltpu.sync_copy(d_vmem, o_hbm.at[i_vmem.at[0]])

        pltpu.emit_pipeline(
            body,
            grid=(2 * _N // _W,),
            in_specs=[
                pl.BlockSpec((_W, _DH), lambda i: (jax.lax.rem(i, nsrc), 0)),
                pl.BlockSpec(
                    (1, _W), lambda i: (i // nlane, jax.lax.rem(i, nlane))),
            ],
            out_specs=[],
            core_axis_name=("c", "s"),
            dimension_semantics=(pltpu.PARALLEL,),
        )(d_hbm, i_hbm)

    return k(data, idx)


def _sc_gather(data, idx, nrows):
    nlane = idx.shape[1] // _W

    @pl.kernel(out_type=jax.ShapeDtypeStruct((nrows, _DH), data.dtype),
               mesh=_vmesh())
    def k(d_hbm, i_hbm, o_hbm):
        def body(i_vmem, o_vmem):
            pltpu.sync_copy(d_hbm.at[i_vmem.at[0]], o_vmem)

        pltpu.emit_pipeline(
            body,
            grid=(nrows // _W,),
            in_specs=[pl.BlockSpec(
                (1, _W), lambda i: (i // nlane, jax.lax.rem(i, nlane)))],
            out_specs=[pl.BlockSpec((_W, _DH), lambda i: (i, 0))],
            core_axis_name=("c", "s"),
            dimension_semantics=(pltpu.PARALLEL,),
        )(i_hbm, o_hbm)

    return k(data, idx)


@jax.jit
def kernel(x, moment, W1, b1, W2, b2, Wg, bg, ln_w, ln_b):
    f32, i32 = jnp.float32, jnp.int32
    posq, p1, p2, gb, nblk = pl.pallas_call(
        _route_meta_body,
        grid=(1,),
        in_specs=[
            pl.BlockSpec((_N, _D), lambda i: (0, 0)),
            pl.BlockSpec((_D, _E), lambda i: (0, 0)),
            pl.BlockSpec((1, _E), lambda i: (0, 0)),
        ],
        out_specs=[pl.BlockSpec((_NC, 2 * _CB), lambda i: (0, 0)),
                   pl.BlockSpec((_N, 1), lambda i: (0, 0)),
                   pl.BlockSpec((_N, 1), lambda i: (0, 0)),
                   pl.BlockSpec((1, 128), lambda i: (0, 0)),
                   pl.BlockSpec((1, 1), lambda i: (0, 0))],
        out_shape=[jax.ShapeDtypeStruct((_NC, 2 * _CB), i32),
                   jax.ShapeDtypeStruct((_N, 1), f32),
                   jax.ShapeDtypeStruct((_N, 1), f32),
                   jax.ShapeDtypeStruct((1, 128), i32),
                   jax.ShapeDtypeStruct((1, 1), i32)],
        scratch_shapes=[pltpu.VMEM((_N, 1), i32),
                        pltpu.VMEM((_N, 1), i32),
                        pltpu.VMEM((_N, 2), f32)],
        compiler_params=pltpu.CompilerParams(
            dimension_semantics=("arbitrary",)),
    )(x, Wg, bg.reshape(1, _E))

    xlo = jax.lax.slice(x, (0, 0), (_N, _DH))
    xhi = jax.lax.slice(x, (0, _DH), (_N, _D))
    xslo = _sc_scatter(xlo, posq)
    xshi = _sc_scatter(xhi, posq)

    def _clamped(b, nb):
        return jnp.minimum(b, nb[0] - 1)

    yshp = jax.ShapeDtypeStruct((_PMAX, _DH), f32)
    yslo, yshi = pl.pallas_call(
        _group_mm_body,
        grid_spec=pltpu.PrefetchScalarGridSpec(
            num_scalar_prefetch=2,
            grid=(_NBLK,),
            in_specs=[
                pl.BlockSpec((_BLK, _DH),
                             lambda b, gb, nb: (_clamped(b, nb), 0)),
                pl.BlockSpec((_BLK, _DH),
                             lambda b, gb, nb: (_clamped(b, nb), 0)),
                pl.BlockSpec((1, _D, _H),
                             lambda b, gb, nb: (gb[_clamped(b, nb)], 0, 0)),
                pl.BlockSpec((1, 1, _H),
                             lambda b, gb, nb: (gb[_clamped(b, nb)], 0, 0)),
                pl.BlockSpec((1, _H, _D),
                             lambda b, gb, nb: (gb[_clamped(b, nb)], 0, 0)),
                pl.BlockSpec((1, 1, _D),
                             lambda b, gb, nb: (gb[_clamped(b, nb)], 0, 0)),
            ],
            out_specs=[
                pl.BlockSpec((_BLK, _DH),
                             lambda b, gb, nb: (_clamped(b, nb), 0)),
                pl.BlockSpec((_BLK, _DH),
                             lambda b, gb, nb: (_clamped(b, nb), 0)),
            ],
        ),
        out_shape=[yshp, yshp],
        compiler_params=pltpu.CompilerParams(
            dimension_semantics=("arbitrary",)),
    )(gb.reshape(128), nblk.reshape(1), xslo, xshi, W1,
      b1.reshape(_E, 1, _H), W2, b2.reshape(_E, 1, _D))

    y01lo = _sc_gather(yslo, posq, 2 * _N)
    y01hi = _sc_gather(yshi, posq, 2 * _N)

    bn = 512
    shp = jax.ShapeDtypeStruct((_N, _D), f32)
    out, p, v, mm = pl.pallas_call(
        _final_body,
        grid=(_N // bn,),
        in_specs=[
            pl.BlockSpec((bn, _DH), lambda n: (n, 0)),
            pl.BlockSpec((bn, _DH), lambda n: (n, 0)),
            pl.BlockSpec((bn, _DH), lambda n: (n + _N // bn, 0)),
            pl.BlockSpec((bn, _DH), lambda n: (n + _N // bn, 0)),
            pl.BlockSpec((bn, 1), lambda n: (n, 0)),
            pl.BlockSpec((bn, 1), lambda n: (n, 0)),
            pl.BlockSpec((3, bn, _D), lambda n: (0, n, 0)),
            pl.BlockSpec((1, _D), lambda n: (0, 0)),
            pl.BlockSpec((1, _D), lambda n: (0, 0)),
        ],
        out_specs=[pl.BlockSpec((bn, _D), lambda n: (n, 0))] * 4,
        out_shape=[shp, shp, shp, shp],
    )(y01lo, y01hi, y01lo, y01hi, p1, p2, moment,
      ln_w.reshape(1, _D), ln_b.reshape(1, _D))
    return (out, p, v, mm)
